# baseline (speedup 1.0000x reference)
import jax
import jax.numpy as jnp
from jax.experimental import pallas as pl
from jax.experimental.pallas import tpu as pltpu

N_TOKENS = 16384
D_MODEL = 4096
N_EXPERTS = 64
TOP_K = 8
NBUF = 2
BUF_ROWS = 1024
CHUNK_SIZES = (256, 256) + (1024,) * 15 + (256, 256)
CHUNK_OFFS = []
_o = 0
for _s in CHUNK_SIZES:
    CHUNK_OFFS.append(_o)
    _o += _s
assert _o == N_TOKENS


def _route_chunk(xb, wt, sz):
    logits = jax.lax.dot_general(
        xb, wt, (((1,), (0,)), ((), ())),
        preferred_element_type=jnp.float32,
    )
    m = jnp.max(logits, axis=1, keepdims=True)
    e = jnp.exp(logits - m)
    p = e * (1.0 / jnp.sum(e, axis=1, keepdims=True))

    cols = jax.lax.broadcasted_iota(jnp.int32, (sz, N_EXPERTS), 1)
    bits = jax.lax.bitcast_convert_type(p, jnp.int32)
    packed = (bits & ~0x3F) | (N_EXPERTS - 1 - cols)

    sentinel = jnp.int32(-0x80000000)
    tops = []
    work = packed
    for _ in range(TOP_K):
        mx = jnp.max(work, axis=1, keepdims=True)
        tops.append(mx)
        work = jnp.where(work == mx, sentinel, work)
    top = jnp.concatenate(tops, axis=1)
    idxs = (N_EXPERTS - 1) - (top & 0x3F)
    vals = jax.lax.bitcast_convert_type(top & ~0x3F, jnp.float32)
    contrib = jnp.sum((work < 0).astype(jnp.int32), axis=0,
                      keepdims=True)
    return vals, idxs, contrib


def _router_body(x_hbm, wt_ref, w_out, idx_out, cnt_ref, buf, sems):
    def copy(c, slot):
        sz, off = CHUNK_SIZES[c], CHUNK_OFFS[c]
        return pltpu.make_async_copy(
            x_hbm.at[pl.ds(off, sz), :],
            buf.at[slot, pl.ds(0, sz), :],
            sems.at[slot])

    cnt_ref[...] = jnp.zeros_like(cnt_ref)
    wt = wt_ref[...]
    for c in range(NBUF):
        copy(c, c).start()
    for c in range(len(CHUNK_SIZES)):
        slot = c % NBUF
        copy(c, slot).wait()
        sz, off = CHUNK_SIZES[c], CHUNK_OFFS[c]
        vals, idxs, contrib = _route_chunk(buf[slot, 0:sz, :], wt, sz)
        w_out[pl.ds(off, sz), :] = vals
        idx_out[pl.ds(off, sz), :] = idxs
        cnt_ref[...] += contrib
        if c + NBUF < len(CHUNK_SIZES):
            copy(c + NBUF, slot).start()


def kernel(x, W):
    n, d = x.shape
    e = W.shape[0]
    wt = W.T
    weights, indices, counts = pl.pallas_call(
        _router_body,
        in_specs=[
            pl.BlockSpec(memory_space=pltpu.MemorySpace.HBM),
            pl.BlockSpec(memory_space=pltpu.MemorySpace.VMEM),
        ],
        out_specs=[
            pl.BlockSpec(memory_space=pltpu.MemorySpace.VMEM),
            pl.BlockSpec(memory_space=pltpu.MemorySpace.VMEM),
            pl.BlockSpec(memory_space=pltpu.MemorySpace.VMEM),
        ],
        out_shape=[
            jax.ShapeDtypeStruct((n, TOP_K), jnp.float32),
            jax.ShapeDtypeStruct((n, TOP_K), jnp.int32),
            jax.ShapeDtypeStruct((1, e), jnp.int32),
        ],
        scratch_shapes=[
            pltpu.VMEM((NBUF, BUF_ROWS, d), jnp.float32),
            pltpu.SemaphoreType.DMA((NBUF,)),
        ],
    )(x, wt)
    return (weights.astype(x.dtype), indices.astype(jnp.int64),
            counts.reshape(e))
